# baseline (speedup 1.0000x reference)
import functools

import jax
import jax.numpy as jnp
from jax import lax
from jax.experimental import pallas as pl
from jax.experimental.pallas import tpu as pltpu
from jax.experimental.pallas import tpu_sc as plsc

_N = 10000
_E = 320000
_D = 128
_H = 64
_DE = 16

_NC = 2
_NS = 16
_NW = _NC * _NS
_EPT = _E // _NW
_CHUNK = 80
_NCHUNK = _EPT // _CHUNK
_RPT = 624
_RTAIL = _N - _NS * _RPT
_NBUF = 5


def _tc1_body(x_ref, wl_ref, bl_ref, wr_ref, p_ref, s_ref):
    x = x_ref[...]
    p_ref[...] = jnp.dot(x, wl_ref[...], preferred_element_type=jnp.float32)
    s_ref[...] = (jnp.dot(x, wr_ref[...], preferred_element_type=jnp.float32)
                  + bl_ref[...])


def _tc_dense1(x, Wl0, bl0, Wr0):
    bn = 2000
    return pl.pallas_call(
        _tc1_body,
        grid=(_N // bn,),
        in_specs=[
            pl.BlockSpec((bn, _D), lambda i: (i, 0)),
            pl.BlockSpec((_D, _H), lambda i: (0, 0)),
            pl.BlockSpec((1, _H), lambda i: (0, 0)),
            pl.BlockSpec((_D, _H), lambda i: (0, 0)),
        ],
        out_specs=[
            pl.BlockSpec((bn, _H), lambda i: (i, 0)),
            pl.BlockSpec((bn, _H), lambda i: (i, 0)),
        ],
        out_shape=[
            jax.ShapeDtypeStruct((_N, _H), jnp.float32),
            jax.ShapeDtypeStruct((_N, _H), jnp.float32),
        ],
    )(x, Wl0, bl0.reshape(1, _H), Wr0)


def _tc_mid_body(p_ref, c_ref, s_ref, wl_ref, bl_ref, wr_ref, po_ref, so_ref):
    cnt = c_ref[0, :, 0:1] + c_ref[1, :, 0:1]
    inv = 1.0 / jnp.maximum(cnt, 1.0)
    psum = p_ref[:, pl.ds(0, _H)] + p_ref[:, pl.ds(_H, _H)]
    h = jnp.maximum(psum * inv + s_ref[...], 0.0)
    po_ref[...] = jnp.dot(h, wl_ref[...], preferred_element_type=jnp.float32)
    so_ref[...] = (jnp.dot(h, wr_ref[...], preferred_element_type=jnp.float32)
                   + bl_ref[...])


def _tc_mid(part, cnt, s0, Wl1, bl1, Wr1):
    bn = 2000
    return pl.pallas_call(
        _tc_mid_body,
        grid=(_N // bn,),
        in_specs=[
            pl.BlockSpec((bn, _NC * _H), lambda i: (i, 0)),
            pl.BlockSpec((_NC, bn, 16), lambda i: (0, i, 0)),
            pl.BlockSpec((bn, _H), lambda i: (i, 0)),
            pl.BlockSpec((_H, _H), lambda i: (0, 0)),
            pl.BlockSpec((1, _H), lambda i: (0, 0)),
            pl.BlockSpec((_H, _H), lambda i: (0, 0)),
        ],
        out_specs=[
            pl.BlockSpec((bn, _H), lambda i: (i, 0)),
            pl.BlockSpec((bn, _H), lambda i: (i, 0)),
        ],
        out_shape=[
            jax.ShapeDtypeStruct((_N, _H), jnp.float32),
            jax.ShapeDtypeStruct((_N, _H), jnp.float32),
        ],
    )(part, cnt, s0, Wl1, bl1.reshape(1, _H), Wr1)


def _tc_fin_body(p_ref, c_ref, s_ref, wa_ref, wb_ref, a_ref, b_ref):
    cnt = c_ref[0, :, 0:1] + c_ref[1, :, 0:1]
    inv = 1.0 / jnp.maximum(cnt, 1.0)
    psum = p_ref[:, pl.ds(0, _H)] + p_ref[:, pl.ds(_H, _H)]
    h = jnp.maximum(psum * inv + s_ref[...], 0.0)
    a_ref[...] = jnp.dot(h, wa_ref[...], preferred_element_type=jnp.float32)
    b_ref[...] = jnp.dot(h, wb_ref[...], preferred_element_type=jnp.float32)


def _tc_fin(part, cnt, s1, W1a, W1b):
    bn = 2000
    return pl.pallas_call(
        _tc_fin_body,
        grid=(_N // bn,),
        in_specs=[
            pl.BlockSpec((bn, _NC * _H), lambda i: (i, 0)),
            pl.BlockSpec((_NC, bn, 16), lambda i: (0, i, 0)),
            pl.BlockSpec((bn, _H), lambda i: (i, 0)),
            pl.BlockSpec((_H, _H), lambda i: (0, 0)),
            pl.BlockSpec((_H, _H), lambda i: (0, 0)),
        ],
        out_specs=[
            pl.BlockSpec((bn, _H), lambda i: (i, 0)),
            pl.BlockSpec((bn, _H), lambda i: (i, 0)),
        ],
        out_shape=[
            jax.ShapeDtypeStruct((_N, _H), jnp.float32),
            jax.ShapeDtypeStruct((_N, _H), jnp.float32),
        ],
    )(part, cnt, s1, W1a, W1b)


def _tc_ea_body(ea_ref, eb_ref, w_ref, b_ref, o_ref):
    o_ref[:, pl.ds(0, _H)] = (
        jnp.dot(ea_ref[...], w_ref[...], preferred_element_type=jnp.float32)
        + b_ref[...])
    o_ref[:, pl.ds(_H, _H)] = (
        jnp.dot(eb_ref[...], w_ref[...], preferred_element_type=jnp.float32)
        + b_ref[...])


def _tc_ea(edge_attr, W1c, b1):
    be = 4000
    return pl.pallas_call(
        _tc_ea_body,
        grid=(_E // (2 * be),),
        in_specs=[
            pl.BlockSpec((be, _DE), lambda i: (2 * i, 0)),
            pl.BlockSpec((be, _DE), lambda i: (2 * i + 1, 0)),
            pl.BlockSpec((_DE, _H), lambda i: (0, 0)),
            pl.BlockSpec((1, _H), lambda i: (0, 0)),
        ],
        out_specs=pl.BlockSpec((be, 2 * _H), lambda i: (i, 0)),
        out_shape=jax.ShapeDtypeStruct((_E // 2, 2 * _H), jnp.float32),
    )(edge_attr, edge_attr, W1c, b1.reshape(1, _H))


def _seg_call(p, src3, dst3, with_counts):
    mesh = plsc.VectorSubcoreMesh(core_axis_name="c", subcore_axis_name="s")
    out_type = [jax.ShapeDtypeStruct((_N, _NC * _H), jnp.float32)]
    scratch = [
        pltpu.VMEM((_NCHUNK, _CHUNK), jnp.int32),
        pltpu.VMEM((_NCHUNK, _CHUNK), jnp.int32),
    ] + [pltpu.VMEM((_CHUNK, _H), jnp.float32) for _ in range(_NBUF)] + [
        pltpu.VMEM_SHARED((_N, _H), jnp.float32),
    ] + [pltpu.SemaphoreType.DMA for _ in range(2 * _NBUF)]
    if with_counts:
        out_type.append(jax.ShapeDtypeStruct((_NC, _N, 16), jnp.float32))
        scratch += [
            pltpu.VMEM((_CHUNK, 16), jnp.float32),
            pltpu.VMEM_SHARED((_N, 16), jnp.float32),
        ]

    def body(p_hbm, src_hbm, dst_hbm, z64_hbm, z16_hbm, *refs):
        if with_counts:
            out_hbm, cnt_hbm = refs[0], refs[1]
            refs = refs[2:]
        else:
            out_hbm = refs[0]
            refs = refs[1:]
        sidx, didx = refs[0], refs[1]
        rows = refs[2:2 + _NBUF]
        acc_sh = refs[2 + _NBUF]
        sem_g = refs[3 + _NBUF:3 + 2 * _NBUF]
        sem_s = refs[3 + 2 * _NBUF:3 + 3 * _NBUF]
        if with_counts:
            ones, cnt_sh = refs[3 + 3 * _NBUF], refs[4 + 3 * _NBUF]
        cid = lax.axis_index("c")
        sid = lax.axis_index("s")
        wid = sid * _NC + cid
        row0 = sid * _RPT

        pltpu.sync_copy(z64_hbm.at[pl.ds(row0, _RPT)],
                        acc_sh.at[pl.ds(row0, _RPT)])
        if with_counts:
            pltpu.sync_copy(z16_hbm.at[pl.ds(row0, _RPT)],
                            cnt_sh.at[pl.ds(row0, _RPT)])
            ov = jnp.ones((16,), jnp.float32)

            def _orow(i, carry):
                ones[i, pl.ds(0, 16)] = ov
                return carry
            lax.fori_loop(0, _CHUNK, _orow, 0)

        @pl.when(sid == _NS - 1)
        def _init_tail():
            pltpu.sync_copy(z64_hbm.at[pl.ds(_NS * _RPT, _RTAIL)],
                            acc_sh.at[pl.ds(_NS * _RPT, _RTAIL)])
            if with_counts:
                pltpu.sync_copy(z16_hbm.at[pl.ds(_NS * _RPT, _RTAIL)],
                                cnt_sh.at[pl.ds(_NS * _RPT, _RTAIL)])
        pltpu.sync_copy(src_hbm.at[wid], sidx)
        pltpu.sync_copy(dst_hbm.at[wid], didx)
        plsc.subcore_barrier()

        def _fire_gather(c, buf, sem):
            pltpu.async_copy(p_hbm.at[sidx.at[c]], buf, sem)

        def _wait_gather(c, buf, sem):
            pltpu.make_async_copy(p_hbm.at[sidx.at[c]], buf, sem).wait()

        def _fire_scatter(c, buf, sem):
            pltpu.async_copy(buf, acc_sh.at[didx.at[c]], sem, add=True)
            if with_counts:
                pltpu.async_copy(ones, cnt_sh.at[didx.at[c]], sem, add=True)

        def _wait_scatter(c, buf, sem):
            pltpu.make_async_copy(buf, acc_sh.at[didx.at[c]], sem).wait()
            if with_counts:
                pltpu.make_async_copy(ones, cnt_sh.at[didx.at[c]], sem).wait()

        for k in range(_NBUF):
            _fire_gather(k, rows[k], sem_g[k])

        def _round(i, carry):
            c0 = i * _NBUF
            for k in range(_NBUF):
                _wait_gather(c0 + k, rows[k], sem_g[k])
                _fire_scatter(c0 + k, rows[k], sem_s[k])
            for k in range(_NBUF):
                _wait_scatter(c0 + k, rows[k], sem_s[k])

                @pl.when(c0 + k + _NBUF < _NCHUNK)
                def _():
                    _fire_gather(c0 + k + _NBUF, rows[k], sem_g[k])
            return carry
        lax.fori_loop(0, _NCHUNK // _NBUF, _round, 0)
        plsc.subcore_barrier()

        coff = cid * _H
        pltpu.sync_copy(acc_sh.at[pl.ds(row0, _RPT)],
                        out_hbm.at[pl.ds(row0, _RPT), pl.ds(coff, _H)])
        if with_counts:
            pltpu.sync_copy(cnt_sh.at[pl.ds(row0, _RPT)],
                            cnt_hbm.at[cid, pl.ds(row0, _RPT)])

        @pl.when(sid == _NS - 1)
        def _out_tail():
            pltpu.sync_copy(acc_sh.at[pl.ds(_NS * _RPT, _RTAIL)],
                            out_hbm.at[pl.ds(_NS * _RPT, _RTAIL),
                                       pl.ds(coff, _H)])
            if with_counts:
                pltpu.sync_copy(cnt_sh.at[pl.ds(_NS * _RPT, _RTAIL)],
                                cnt_hbm.at[cid, pl.ds(_NS * _RPT, _RTAIL)])

    f = pl.kernel(body, out_type=tuple(out_type), mesh=mesh,
                  scratch_types=scratch,
                  compiler_params=pltpu.CompilerParams(
                      use_tc_tiling_on_sc=False,
                      needs_layout_passes=False))
    z64 = jnp.zeros((_N, _H), jnp.float32)
    z16 = jnp.zeros((_N, 16), jnp.float32)
    return f(p, src3, dst3, z64, z16)


def _edge_call(a, b, ea, src3, dst3, w2mat, b2vec):
    mesh = plsc.VectorSubcoreMesh(core_axis_name="c", subcore_axis_name="s")
    scratch = [
        pltpu.VMEM((_NCHUNK, _CHUNK), jnp.int32),
        pltpu.VMEM((_NCHUNK, _CHUNK), jnp.int32),
        pltpu.VMEM((_CHUNK, _H), jnp.float32),
        pltpu.VMEM((_CHUNK, _H), jnp.float32),
        pltpu.VMEM((_CHUNK, _H), jnp.float32),
        pltpu.VMEM((_CHUNK, _H), jnp.float32),
        pltpu.VMEM((_CHUNK, _H), jnp.float32),
        pltpu.VMEM((_CHUNK, _H), jnp.float32),
        pltpu.VMEM((_CHUNK,), jnp.float32),
        pltpu.VMEM((_CHUNK,), jnp.float32),
        pltpu.VMEM((_H,), jnp.float32),
        pltpu.VMEM((16,), jnp.float32),
        pltpu.VMEM((16, 17), jnp.float32),
        pltpu.SemaphoreType.DMA,
        pltpu.SemaphoreType.DMA,
        pltpu.SemaphoreType.DMA,
        pltpu.SemaphoreType.DMA,
    ]

    def body(a_hbm, b_hbm, ea_hbm, src_hbm, dst_hbm, w2_hbm, b2_hbm, out_hbm,
             sidx, didx, ar0, ar1, br0, br1, er0, er1, ov0, ov1, w2v, b2v,
             pbuf, sem_g0, sem_g1, sem_w0, sem_w1):
        cid = lax.axis_index("c")
        sid = lax.axis_index("s")
        wid = sid * _NC + cid
        pltpu.sync_copy(w2_hbm, w2v)
        pltpu.sync_copy(b2_hbm, b2v)
        pltpu.sync_copy(src_hbm.at[wid], sidx)
        pltpu.sync_copy(dst_hbm.at[wid], didx)
        b2r = b2v[pl.ds(0, 16)]
        zr = jnp.zeros((16,), jnp.float32)
        iota = lax.iota(jnp.int32, 16)
        e0 = wid * _EPT

        def _ea_slice(c):
            base = e0 + c * _CHUNK
            rem = lax.rem(base, 8000)
            r0 = (base // 8000) * 4000 + lax.rem(rem, 4000)
            coff = (rem // 4000) * _H
            return ea_hbm.at[pl.ds(r0, _CHUNK), pl.ds(coff, _H)]

        def _fire(c, ar, br, er, sem):
            pltpu.async_copy(a_hbm.at[sidx.at[c]], ar, sem)
            pltpu.async_copy(b_hbm.at[didx.at[c]], br, sem)
            pltpu.async_copy(_ea_slice(c), er, sem)

        def _wait(c, ar, br, er, sem):
            pltpu.make_async_copy(a_hbm.at[sidx.at[c]], ar, sem).wait()
            pltpu.make_async_copy(b_hbm.at[didx.at[c]], br, sem).wait()
            pltpu.make_async_copy(_ea_slice(c), er, sem).wait()

        def _wait_write(c, ov, sem):
            base = e0 + c * _CHUNK
            pltpu.make_async_copy(ov, out_hbm.at[pl.ds(base, _CHUNK)],
                                  sem).wait()

        ngrp = _CHUNK // 16
        w2q = [w2v[pl.ds(q * 16, 16)] for q in range(_H // 16)]

        def _compute(arows, brows, erows, outv):
            for g in range(ngrp):
                def _estep(el, carry):
                    e = el + g * 16
                    pa = None
                    for q in range(_H // 16):
                        av = arows[e, pl.ds(q * 16, 16)]
                        bv = brows[e, pl.ds(q * 16, 16)]
                        ev = erows[e, pl.ds(q * 16, 16)]
                        h = jnp.maximum(av + bv + ev, 0.0)
                        t = h * w2q[q]
                        pa = t if pa is None else pa + t
                    plsc.store_scatter(
                        pbuf, [iota, jnp.full((16,), el, jnp.int32)], pa)
                    return carry
                lax.fori_loop(0, 16, _estep, 0, unroll=4)
                acc0 = b2r
                acc1 = zr
                for i in range(0, 16, 2):
                    acc0 = acc0 + pbuf[i, pl.ds(0, 16)]
                    acc1 = acc1 + pbuf[i + 1, pl.ds(0, 16)]
                outv[pl.ds(g * 16, 16)] = acc0 + acc1

        _fire(0, ar0, br0, er0, sem_g0)
        _fire(1, ar1, br1, er1, sem_g1)

        def _pair(g, carry):
            i0 = 2 * g
            i1 = i0 + 1
            _wait(i0, ar0, br0, er0, sem_g0)

            @pl.when(g > 0)
            def _():
                _wait_write(i0 - 2, ov0, sem_w0)
            _compute(ar0, br0, er0, ov0)
            pltpu.async_copy(ov0, out_hbm.at[pl.ds(e0 + i0 * _CHUNK, _CHUNK)],
                             sem_w0)
            _fire(i0 + 2, ar0, br0, er0, sem_g0)
            _wait(i1, ar1, br1, er1, sem_g1)

            @pl.when(g > 0)
            def _():
                _wait_write(i1 - 2, ov1, sem_w1)
            _compute(ar1, br1, er1, ov1)
            pltpu.async_copy(ov1, out_hbm.at[pl.ds(e0 + i1 * _CHUNK, _CHUNK)],
                             sem_w1)

            @pl.when(g < _NCHUNK // 2 - 1)
            def _():
                _fire(i1 + 2, ar1, br1, er1, sem_g1)
            return carry
        lax.fori_loop(0, _NCHUNK // 2, _pair, 0)
        ct = _NCHUNK - 1
        _wait(ct, ar0, br0, er0, sem_g0)
        _wait_write(ct - 2, ov0, sem_w0)
        _compute(ar0, br0, er0, ov0)
        pltpu.sync_copy(ov0, out_hbm.at[pl.ds(e0 + ct * _CHUNK, _CHUNK)])
        _wait_write(ct - 1, ov1, sem_w1)

    f = pl.kernel(body, out_type=jax.ShapeDtypeStruct((_E,), jnp.float32),
                  mesh=mesh, scratch_types=scratch,
                  compiler_params=pltpu.CompilerParams(
                      use_tc_tiling_on_sc=False,
                      needs_layout_passes=False))
    return f(a, b, ea, src3, dst3, w2mat, b2vec)


def kernel(x, edge_index, edge_attr, Wl0, bl0, Wr0, Wl1, bl1, Wr1, W1, b1,
           W2, b2):
    src3 = edge_index[0].astype(jnp.int32).reshape(_NW, _NCHUNK, _CHUNK)
    dst3 = edge_index[1].astype(jnp.int32).reshape(_NW, _NCHUNK, _CHUNK)
    ea = _tc_ea(edge_attr, W1[2 * _H:], b1)
    p0, s0 = _tc_dense1(x, Wl0, bl0, Wr0)
    part0, cnt = _seg_call(p0, src3, dst3, with_counts=True)
    p1, s1 = _tc_mid(part0, cnt, s0, Wl1, bl1, Wr1)
    (part1,) = _seg_call(p1, src3, dst3, with_counts=False)
    a, bmat = _tc_fin(part1, cnt, s1, W1[:_H], W1[_H:2 * _H])
    w2vec = W2.reshape(_H)
    b2vec = jnp.broadcast_to(b2.reshape(1), (16,))
    return _edge_call(a, bmat, ea, src3, dst3, w2vec, b2vec)


# speedup vs baseline: 1.0003x; 1.0003x over previous
import jax
import jax.numpy as jnp
from jax import lax
from jax.experimental import pallas as pl
from jax.experimental.pallas import tpu as pltpu
from jax.experimental.pallas import tpu_sc as plsc

_N = 10000
_E = 320000
_D = 128
_H = 64
_DE = 16

_NC = 2
_NS = 16
_NW = _NC * _NS
_EPT = _E // _NW
_CHUNK = 80
_NCHUNK = _EPT // _CHUNK
_RPT = 624
_RTAIL = _N - _NS * _RPT
_NBUF = 5


def _tc1_body(x_ref, wl_ref, bl_ref, wr_ref, p_ref, s_ref):
    x = x_ref[...]
    p_ref[...] = jnp.dot(x, wl_ref[...], preferred_element_type=jnp.float32)
    s_ref[...] = (jnp.dot(x, wr_ref[...], preferred_element_type=jnp.float32)
                  + bl_ref[...])


def _tc_dense1(x, Wl0, bl0, Wr0):
    bn = 2000
    return pl.pallas_call(
        _tc1_body,
        grid=(_N // bn,),
        in_specs=[
            pl.BlockSpec((bn, _D), lambda i: (i, 0)),
            pl.BlockSpec((_D, _H), lambda i: (0, 0)),
            pl.BlockSpec((1, _H), lambda i: (0, 0)),
            pl.BlockSpec((_D, _H), lambda i: (0, 0)),
        ],
        out_specs=[
            pl.BlockSpec((bn, _H), lambda i: (i, 0)),
            pl.BlockSpec((bn, _H), lambda i: (i, 0)),
        ],
        out_shape=[
            jax.ShapeDtypeStruct((_N, _H), jnp.float32),
            jax.ShapeDtypeStruct((_N, _H), jnp.float32),
        ],
    )(x, Wl0, bl0.reshape(1, _H), Wr0)


def _tc_mid_body(p_ref, c_ref, s_ref, wl_ref, bl_ref, wr_ref, po_ref, so_ref):
    cnt = c_ref[0, :, 0:1] + c_ref[1, :, 0:1]
    inv = 1.0 / jnp.maximum(cnt, 1.0)
    psum = p_ref[:, pl.ds(0, _H)] + p_ref[:, pl.ds(_H, _H)]
    h = jnp.maximum(psum * inv + s_ref[...], 0.0)
    po_ref[...] = jnp.dot(h, wl_ref[...], preferred_element_type=jnp.float32)
    so_ref[...] = (jnp.dot(h, wr_ref[...], preferred_element_type=jnp.float32)
                   + bl_ref[...])


def _tc_mid(part, cnt, s0, Wl1, bl1, Wr1):
    bn = 2000
    return pl.pallas_call(
        _tc_mid_body,
        grid=(_N // bn,),
        in_specs=[
            pl.BlockSpec((bn, _NC * _H), lambda i: (i, 0)),
            pl.BlockSpec((_NC, bn, 16), lambda i: (0, i, 0)),
            pl.BlockSpec((bn, _H), lambda i: (i, 0)),
            pl.BlockSpec((_H, _H), lambda i: (0, 0)),
            pl.BlockSpec((1, _H), lambda i: (0, 0)),
            pl.BlockSpec((_H, _H), lambda i: (0, 0)),
        ],
        out_specs=[
            pl.BlockSpec((bn, _H), lambda i: (i, 0)),
            pl.BlockSpec((bn, _H), lambda i: (i, 0)),
        ],
        out_shape=[
            jax.ShapeDtypeStruct((_N, _H), jnp.float32),
            jax.ShapeDtypeStruct((_N, _H), jnp.float32),
        ],
    )(part, cnt, s0, Wl1, bl1.reshape(1, _H), Wr1)


def _tc_fin_body(p_ref, c_ref, s_ref, wa_ref, wb_ref, a_ref, b_ref):
    cnt = c_ref[0, :, 0:1] + c_ref[1, :, 0:1]
    inv = 1.0 / jnp.maximum(cnt, 1.0)
    psum = p_ref[:, pl.ds(0, _H)] + p_ref[:, pl.ds(_H, _H)]
    h = jnp.maximum(psum * inv + s_ref[...], 0.0)
    a_ref[...] = jnp.dot(h, wa_ref[...], preferred_element_type=jnp.float32)
    b_ref[...] = jnp.dot(h, wb_ref[...], preferred_element_type=jnp.float32)


def _tc_fin(part, cnt, s1, W1a, W1b):
    bn = 2000
    return pl.pallas_call(
        _tc_fin_body,
        grid=(_N // bn,),
        in_specs=[
            pl.BlockSpec((bn, _NC * _H), lambda i: (i, 0)),
            pl.BlockSpec((_NC, bn, 16), lambda i: (0, i, 0)),
            pl.BlockSpec((bn, _H), lambda i: (i, 0)),
            pl.BlockSpec((_H, _H), lambda i: (0, 0)),
            pl.BlockSpec((_H, _H), lambda i: (0, 0)),
        ],
        out_specs=[
            pl.BlockSpec((bn, _H), lambda i: (i, 0)),
            pl.BlockSpec((bn, _H), lambda i: (i, 0)),
        ],
        out_shape=[
            jax.ShapeDtypeStruct((_N, _H), jnp.float32),
            jax.ShapeDtypeStruct((_N, _H), jnp.float32),
        ],
    )(part, cnt, s1, W1a, W1b)


def _tc_ea_body(ea_ref, eb_ref, w_ref, b_ref, o_ref):
    o_ref[:, pl.ds(0, _H)] = (
        jnp.dot(ea_ref[...], w_ref[...], preferred_element_type=jnp.float32)
        + b_ref[...])
    o_ref[:, pl.ds(_H, _H)] = (
        jnp.dot(eb_ref[...], w_ref[...], preferred_element_type=jnp.float32)
        + b_ref[...])


def _tc_ea(edge_attr, W1c, b1):
    be = 4000
    return pl.pallas_call(
        _tc_ea_body,
        grid=(_E // (2 * be),),
        in_specs=[
            pl.BlockSpec((be, _DE), lambda i: (2 * i, 0)),
            pl.BlockSpec((be, _DE), lambda i: (2 * i + 1, 0)),
            pl.BlockSpec((_DE, _H), lambda i: (0, 0)),
            pl.BlockSpec((1, _H), lambda i: (0, 0)),
        ],
        out_specs=pl.BlockSpec((be, 2 * _H), lambda i: (i, 0)),
        out_shape=jax.ShapeDtypeStruct((_E // 2, 2 * _H), jnp.float32),
    )(edge_attr, edge_attr, W1c, b1.reshape(1, _H))


def _seg_call(p, src3, dst3, with_counts):
    mesh = plsc.VectorSubcoreMesh(core_axis_name="c", subcore_axis_name="s")
    out_type = [jax.ShapeDtypeStruct((_N, _NC * _H), jnp.float32)]
    scratch = [
        pltpu.VMEM((_NCHUNK, _CHUNK), jnp.int32),
        pltpu.VMEM((_NCHUNK, _CHUNK), jnp.int32),
    ] + [pltpu.VMEM((_CHUNK, _H), jnp.float32) for _ in range(_NBUF)] + [
        pltpu.VMEM_SHARED((_N, _H), jnp.float32),
    ] + [pltpu.SemaphoreType.DMA for _ in range(2 * _NBUF)]
    if with_counts:
        out_type.append(jax.ShapeDtypeStruct((_NC, _N, 16), jnp.float32))
        scratch += [
            pltpu.VMEM((_CHUNK, 16), jnp.float32),
            pltpu.VMEM_SHARED((_N, 16), jnp.float32),
        ]

    def body(p_hbm, src_hbm, dst_hbm, z64_hbm, z16_hbm, *refs):
        if with_counts:
            out_hbm, cnt_hbm = refs[0], refs[1]
            refs = refs[2:]
        else:
            out_hbm = refs[0]
            refs = refs[1:]
        sidx, didx = refs[0], refs[1]
        rows = refs[2:2 + _NBUF]
        acc_sh = refs[2 + _NBUF]
        sem_g = refs[3 + _NBUF:3 + 2 * _NBUF]
        sem_s = refs[3 + 2 * _NBUF:3 + 3 * _NBUF]
        if with_counts:
            ones, cnt_sh = refs[3 + 3 * _NBUF], refs[4 + 3 * _NBUF]
        cid = lax.axis_index("c")
        sid = lax.axis_index("s")
        wid = sid * _NC + cid
        row0 = sid * _RPT

        pltpu.sync_copy(z64_hbm.at[pl.ds(row0, _RPT)],
                        acc_sh.at[pl.ds(row0, _RPT)])
        if with_counts:
            pltpu.sync_copy(z16_hbm.at[pl.ds(row0, _RPT)],
                            cnt_sh.at[pl.ds(row0, _RPT)])
            ov = jnp.ones((16,), jnp.float32)

            def _orow(i, carry):
                ones[i, pl.ds(0, 16)] = ov
                return carry
            lax.fori_loop(0, _CHUNK, _orow, 0)

        @pl.when(sid == _NS - 1)
        def _init_tail():
            pltpu.sync_copy(z64_hbm.at[pl.ds(_NS * _RPT, _RTAIL)],
                            acc_sh.at[pl.ds(_NS * _RPT, _RTAIL)])
            if with_counts:
                pltpu.sync_copy(z16_hbm.at[pl.ds(_NS * _RPT, _RTAIL)],
                                cnt_sh.at[pl.ds(_NS * _RPT, _RTAIL)])
        pltpu.sync_copy(src_hbm.at[wid], sidx)
        pltpu.sync_copy(dst_hbm.at[wid], didx)
        plsc.subcore_barrier()

        def _fire_gather(c, buf, sem):
            pltpu.async_copy(p_hbm.at[sidx.at[c]], buf, sem)

        def _wait_gather(c, buf, sem):
            pltpu.make_async_copy(p_hbm.at[sidx.at[c]], buf, sem).wait()

        def _fire_scatter(c, buf, sem):
            pltpu.async_copy(buf, acc_sh.at[didx.at[c]], sem, add=True)
            if with_counts:
                pltpu.async_copy(ones, cnt_sh.at[didx.at[c]], sem, add=True)

        def _wait_scatter(c, buf, sem):
            pltpu.make_async_copy(buf, acc_sh.at[didx.at[c]], sem).wait()
            if with_counts:
                pltpu.make_async_copy(ones, cnt_sh.at[didx.at[c]], sem).wait()

        for k in range(_NBUF):
            _fire_gather(k, rows[k], sem_g[k])

        def _round(i, carry):
            c0 = i * _NBUF
            for k in range(_NBUF):
                _wait_gather(c0 + k, rows[k], sem_g[k])
                _fire_scatter(c0 + k, rows[k], sem_s[k])
            for k in range(_NBUF):
                _wait_scatter(c0 + k, rows[k], sem_s[k])

                @pl.when(c0 + k + _NBUF < _NCHUNK)
                def _():
                    _fire_gather(c0 + k + _NBUF, rows[k], sem_g[k])
            return carry
        lax.fori_loop(0, _NCHUNK // _NBUF, _round, 0)
        plsc.subcore_barrier()

        coff = cid * _H
        pltpu.sync_copy(acc_sh.at[pl.ds(row0, _RPT)],
                        out_hbm.at[pl.ds(row0, _RPT), pl.ds(coff, _H)])
        if with_counts:
            pltpu.sync_copy(cnt_sh.at[pl.ds(row0, _RPT)],
                            cnt_hbm.at[cid, pl.ds(row0, _RPT)])

        @pl.when(sid == _NS - 1)
        def _out_tail():
            pltpu.sync_copy(acc_sh.at[pl.ds(_NS * _RPT, _RTAIL)],
                            out_hbm.at[pl.ds(_NS * _RPT, _RTAIL),
                                       pl.ds(coff, _H)])
            if with_counts:
                pltpu.sync_copy(cnt_sh.at[pl.ds(_NS * _RPT, _RTAIL)],
                                cnt_hbm.at[cid, pl.ds(_NS * _RPT, _RTAIL)])

    f = pl.kernel(body, out_type=tuple(out_type), mesh=mesh,
                  scratch_types=scratch,
                  compiler_params=pltpu.CompilerParams(
                      use_tc_tiling_on_sc=False,
                      needs_layout_passes=False))
    z64 = jnp.zeros((_N, _H), jnp.float32)
    z16 = jnp.zeros((_N, 16), jnp.float32)
    return f(p, src3, dst3, z64, z16)


def _edge_call(a, b, ea, src3, dst3, w2mat, b2vec):
    mesh = plsc.VectorSubcoreMesh(core_axis_name="c", subcore_axis_name="s")
    scratch = [
        pltpu.VMEM((_NCHUNK, _CHUNK), jnp.int32),
        pltpu.VMEM((_NCHUNK, _CHUNK), jnp.int32),
        pltpu.VMEM((_CHUNK, _H), jnp.float32),
        pltpu.VMEM((_CHUNK, _H), jnp.float32),
        pltpu.VMEM((_CHUNK, _H), jnp.float32),
        pltpu.VMEM((_CHUNK, _H), jnp.float32),
        pltpu.VMEM((_CHUNK, _H), jnp.float32),
        pltpu.VMEM((_CHUNK, _H), jnp.float32),
        pltpu.VMEM((_CHUNK,), jnp.float32),
        pltpu.VMEM((_CHUNK,), jnp.float32),
        pltpu.VMEM((_H,), jnp.float32),
        pltpu.VMEM((16,), jnp.float32),
        pltpu.VMEM((16, 17), jnp.float32),
        pltpu.SemaphoreType.DMA,
        pltpu.SemaphoreType.DMA,
        pltpu.SemaphoreType.DMA,
        pltpu.SemaphoreType.DMA,
    ]

    def body(a_hbm, b_hbm, ea_hbm, src_hbm, dst_hbm, w2_hbm, b2_hbm, out_hbm,
             sidx, didx, ar0, ar1, br0, br1, er0, er1, ov0, ov1, w2v, b2v,
             pbuf, sem_g0, sem_g1, sem_w0, sem_w1):
        cid = lax.axis_index("c")
        sid = lax.axis_index("s")
        wid = sid * _NC + cid
        pltpu.sync_copy(w2_hbm, w2v)
        pltpu.sync_copy(b2_hbm, b2v)
        pltpu.sync_copy(src_hbm.at[wid], sidx)
        pltpu.sync_copy(dst_hbm.at[wid], didx)
        b2r = b2v[pl.ds(0, 16)]
        zr = jnp.zeros((16,), jnp.float32)
        iota = lax.iota(jnp.int32, 16)
        e0 = wid * _EPT

        def _ea_slice(c):
            base = e0 + c * _CHUNK
            rem = lax.rem(base, 8000)
            r0 = (base // 8000) * 4000 + lax.rem(rem, 4000)
            coff = (rem // 4000) * _H
            return ea_hbm.at[pl.ds(r0, _CHUNK), pl.ds(coff, _H)]

        def _fire(c, ar, br, er, sem):
            pltpu.async_copy(a_hbm.at[sidx.at[c]], ar, sem)
            pltpu.async_copy(b_hbm.at[didx.at[c]], br, sem)
            pltpu.async_copy(_ea_slice(c), er, sem)

        def _wait(c, ar, br, er, sem):
            pltpu.make_async_copy(a_hbm.at[sidx.at[c]], ar, sem).wait()
            pltpu.make_async_copy(b_hbm.at[didx.at[c]], br, sem).wait()
            pltpu.make_async_copy(_ea_slice(c), er, sem).wait()

        def _wait_write(c, ov, sem):
            base = e0 + c * _CHUNK
            pltpu.make_async_copy(ov, out_hbm.at[pl.ds(base, _CHUNK)],
                                  sem).wait()

        ngrp = _CHUNK // 16
        w2q = [w2v[pl.ds(q * 16, 16)] for q in range(_H // 16)]

        def _compute(arows, brows, erows, outv):
            for g in range(ngrp):
                def _estep(el, carry):
                    e = el + g * 16
                    pa = None
                    for q in range(_H // 16):
                        av = arows[e, pl.ds(q * 16, 16)]
                        bv = brows[e, pl.ds(q * 16, 16)]
                        ev = erows[e, pl.ds(q * 16, 16)]
                        h = jnp.maximum(av + bv + ev, 0.0)
                        t = h * w2q[q]
                        pa = t if pa is None else pa + t
                    plsc.store_scatter(
                        pbuf, [iota, jnp.full((16,), el, jnp.int32)], pa)
                    return carry
                lax.fori_loop(0, 16, _estep, 0, unroll=4)
                acc0 = b2r
                acc1 = zr
                for i in range(0, 16, 2):
                    acc0 = acc0 + pbuf[i, pl.ds(0, 16)]
                    acc1 = acc1 + pbuf[i + 1, pl.ds(0, 16)]
                outv[pl.ds(g * 16, 16)] = acc0 + acc1

        _fire(0, ar0, br0, er0, sem_g0)
        _fire(1, ar1, br1, er1, sem_g1)

        def _pair(g, carry):
            i0 = 2 * g
            i1 = i0 + 1
            _wait(i0, ar0, br0, er0, sem_g0)

            @pl.when(g > 0)
            def _():
                _wait_write(i0 - 2, ov0, sem_w0)
            _compute(ar0, br0, er0, ov0)
            pltpu.async_copy(ov0, out_hbm.at[pl.ds(e0 + i0 * _CHUNK, _CHUNK)],
                             sem_w0)
            _fire(i0 + 2, ar0, br0, er0, sem_g0)
            _wait(i1, ar1, br1, er1, sem_g1)

            @pl.when(g > 0)
            def _():
                _wait_write(i1 - 2, ov1, sem_w1)
            _compute(ar1, br1, er1, ov1)
            pltpu.async_copy(ov1, out_hbm.at[pl.ds(e0 + i1 * _CHUNK, _CHUNK)],
                             sem_w1)

            @pl.when(g < _NCHUNK // 2 - 1)
            def _():
                _fire(i1 + 2, ar1, br1, er1, sem_g1)
            return carry
        lax.fori_loop(0, _NCHUNK // 2, _pair, 0)
        ct = _NCHUNK - 1
        _wait(ct, ar0, br0, er0, sem_g0)
        _wait_write(ct - 2, ov0, sem_w0)
        _compute(ar0, br0, er0, ov0)
        pltpu.sync_copy(ov0, out_hbm.at[pl.ds(e0 + ct * _CHUNK, _CHUNK)])
        _wait_write(ct - 1, ov1, sem_w1)

    f = pl.kernel(body, out_type=jax.ShapeDtypeStruct((_E,), jnp.float32),
                  mesh=mesh, scratch_types=scratch,
                  compiler_params=pltpu.CompilerParams(
                      use_tc_tiling_on_sc=False,
                      needs_layout_passes=False))
    return f(a, b, ea, src3, dst3, w2mat, b2vec)


def kernel(x, edge_index, edge_attr, Wl0, bl0, Wr0, Wl1, bl1, Wr1, W1, b1,
           W2, b2):
    src3 = edge_index[0].astype(jnp.int32).reshape(_NW, _NCHUNK, _CHUNK)
    dst3 = edge_index[1].astype(jnp.int32).reshape(_NW, _NCHUNK, _CHUNK)
    ea = _tc_ea(edge_attr, W1[2 * _H:], b1)
    p0, s0 = _tc_dense1(x, Wl0, bl0, Wr0)
    part0, cnt = _seg_call(p0, src3, dst3, with_counts=True)
    p1, s1 = _tc_mid(part0, cnt, s0, Wl1, bl1, Wr1)
    (part1,) = _seg_call(p1, src3, dst3, with_counts=False)
    a, bmat = _tc_fin(part1, cnt, s1, W1[:_H], W1[_H:2 * _H])
    w2vec = W2.reshape(_H)
    b2vec = jnp.broadcast_to(b2.reshape(1), (16,))
    return _edge_call(a, bmat, ea, src3, dst3, w2vec, b2vec)
